# baseline (speedup 1.0000x reference)
import functools

import jax
import jax.numpy as jnp
from jax import lax
from jax.experimental import pallas as pl
from jax.experimental.pallas import tpu as pltpu
from jax.experimental.pallas import tpu_sc as plsc

_K = 16
_MM = 8


def _fps_body(xyz_ref, fidx_ref, px_ref, py_ref, pz_ref):
    B, _, S, L = xyz_ref.shape
    N = S * L
    M = N // 4
    x = xyz_ref[:, 0]
    y = xyz_ref[:, 1]
    z = xyz_ref[:, 2]
    iota3 = (lax.broadcasted_iota(jnp.int32, (B, S, L), 1) * L
             + lax.broadcasted_iota(jnp.int32, (B, S, L), 2))
    boff = lax.broadcasted_iota(jnp.int32, (B, 1), 0) * N
    iota128 = lax.broadcasted_iota(jnp.int32, (B, 128), 1)

    def inner(j, st):
        dist, far, cx, cy, cz, pf, ppx, ppy, ppz = st
        lane = iota128 == j
        pf = jnp.where(lane, far[:, 0, :] + boff, pf)
        ppx = jnp.where(lane, cx[:, 0, :], ppx)
        ppy = jnp.where(lane, cy[:, 0, :], ppy)
        ppz = jnp.where(lane, cz[:, 0, :], ppz)
        dx = x - cx
        dy = y - cy
        dz = z - cz
        d = (dx * dx + dy * dy) + dz * dz
        dist = jnp.minimum(dist, d)
        v, ii, xx, yy, zz = dist, iota3, x, y, z
        w = L
        while w > 1:
            h = w // 2
            v1, v2 = v[:, :, :h], v[:, :, h:w]
            i1, i2 = ii[:, :, :h], ii[:, :, h:w]
            cond = (v1 > v2) | ((v1 == v2) & (i1 < i2))
            v = jnp.where(cond, v1, v2)
            ii = jnp.where(cond, i1, i2)
            xx = jnp.where(cond, xx[:, :, :h], xx[:, :, h:w])
            yy = jnp.where(cond, yy[:, :, :h], yy[:, :, h:w])
            zz = jnp.where(cond, zz[:, :, :h], zz[:, :, h:w])
            w = h
        s = S
        while s > 1:
            h = s // 2
            v1, v2 = v[:, :h], v[:, h:s]
            i1, i2 = ii[:, :h], ii[:, h:s]
            cond = (v1 > v2) | ((v1 == v2) & (i1 < i2))
            v = jnp.where(cond, v1, v2)
            ii = jnp.where(cond, i1, i2)
            xx = jnp.where(cond, xx[:, :h], xx[:, h:s])
            yy = jnp.where(cond, yy[:, :h], yy[:, h:s])
            zz = jnp.where(cond, zz[:, :h], zz[:, h:s])
            s = h
        return dist, ii, xx, yy, zz, pf, ppx, ppy, ppz

    def outer(o, carry):
        dist, far, cx, cy, cz = carry
        pf0 = jnp.zeros((B, 128), jnp.int32)
        pp0 = jnp.zeros((B, 128), jnp.float32)
        st = lax.fori_loop(0, 128, inner,
                           (dist, far, cx, cy, cz, pf0, pp0, pp0, pp0))
        dist, far, cx, cy, cz, pf, ppx, ppy, ppz = st
        off = pl.multiple_of(o * 128, 128)
        fidx_ref[:, pl.ds(off, 128)] = pf
        px_ref[:, pl.ds(off, 128)] = ppx
        py_ref[:, pl.ds(off, 128)] = ppy
        pz_ref[:, pl.ds(off, 128)] = ppz
        return dist, far, cx, cy, cz

    dist0 = jnp.full((B, S, L), 1e10, dtype=jnp.float32)
    far0 = jnp.zeros((B, 1, 1), jnp.int32)
    lax.fori_loop(0, M // 128, outer,
                  (dist0, far0, x[:, 0:1, 0:1], y[:, 0:1, 0:1],
                   z[:, 0:1, 0:1]))


def _fps_call(xyz_t):
    B, _, S, L = xyz_t.shape
    M = S * L // 4
    return pl.pallas_call(
        _fps_body,
        out_shape=[
            jax.ShapeDtypeStruct((B, M), jnp.int32),
            jax.ShapeDtypeStruct((B, M), jnp.float32),
            jax.ShapeDtypeStruct((B, M), jnp.float32),
            jax.ShapeDtypeStruct((B, M), jnp.float32),
        ],
    )(xyz_t)


def _knn_body(fm_ref, fmt_ref, kg_ref, km_ref):
    _, RT, C = fm_ref.shape
    _, _, M = fmt_ref.shape
    N = M * 4
    t = fm_ref[0]
    Ft = fmt_ref[0]
    ip = jnp.dot(t, Ft, preferred_element_type=jnp.float32)
    inner = -2.0 * ip
    xxF = jnp.sum(Ft * Ft, axis=0, keepdims=True)
    xxT = jnp.sum(t * t, axis=1, keepdims=True)
    pd = (-xxF - inner) - xxT
    iota2 = lax.broadcasted_iota(jnp.int32, (RT, M), 1)
    b = pl.program_id(0)
    neg = jnp.float32(-jnp.inf)
    for k in range(_K):
        mx = jnp.max(pd, axis=1, keepdims=True)
        idx = jnp.min(jnp.where(pd == mx, iota2, M), axis=1)
        kg_ref[0, k, :] = idx + b * N
        km_ref[0, k, :] = idx + b * M
        pd = jnp.where(iota2 == idx[:, None], neg, pd)


def _knn_call(featM, featM_T, RT=256):
    B, M, C = featM.shape
    grid = (B, M // RT)
    return pl.pallas_call(
        _knn_body,
        grid=grid,
        in_specs=[
            pl.BlockSpec((1, RT, C), lambda b, t: (b, t, 0)),
            pl.BlockSpec((1, C, M), lambda b, t: (b, 0, 0)),
        ],
        out_specs=[
            pl.BlockSpec((1, _K, RT), lambda b, t: (b, 0, t)),
            pl.BlockSpec((1, _K, RT), lambda b, t: (b, 0, t)),
        ],
        out_shape=[
            jax.ShapeDtypeStruct((B, _K, M), jnp.int32),
            jax.ShapeDtypeStruct((B, _K, M), jnp.int32),
        ],
    )(featM, featM_T)


def _sc_gather2(tableA, idxA, tableB, idxB, ch=128):
    D = tableA.shape[1]
    nA = idxA.shape[0]
    nB = idxB.shape[0]
    nw = 32
    perA = nA // nw
    perB = nB // nw
    mesh = plsc.VectorSubcoreMesh(core_axis_name="c", subcore_axis_name="s")

    @functools.partial(
        pl.kernel,
        out_type=[
            jax.ShapeDtypeStruct((nA, D), jnp.float32),
            jax.ShapeDtypeStruct((nB, D), jnp.float32),
        ],
        mesh=mesh,
        scratch_types=[
            pltpu.VMEM((ch,), jnp.int32),
            pltpu.VMEM((ch, D), jnp.float32),
            pltpu.SemaphoreType.DMA,
        ],
        compiler_params=pltpu.CompilerParams(use_tc_tiling_on_sc=False),
    )
    def k(tA, iA, tB, iB, oA, oB, idx_v, rows_v, sem):
        wid = lax.axis_index("s") * 2 + lax.axis_index("c")

        def one(c, tbl, ih, oh, base):
            off = base + c * ch
            pltpu.sync_copy(ih.at[pl.ds(off, ch)], idx_v)
            pltpu.async_copy(tbl.at[idx_v], rows_v, sem).wait()
            pltpu.sync_copy(rows_v, oh.at[pl.ds(off, ch)])

        def bodyA(c, _):
            one(c, tA, iA, oA, wid * perA)
            return 0

        def bodyB(c, _):
            one(c, tB, iB, oB, wid * perB)
            return 0

        lax.fori_loop(0, perA // ch, bodyA, 0)
        lax.fori_loop(0, perB // ch, bodyB, 0)

    return k(tableA, idxA, tableB, idxB)


def _sc_gather1(table, idx, ch=128):
    D = table.shape[1]
    n = idx.shape[0]
    nw = 32
    per = n // nw
    mesh = plsc.VectorSubcoreMesh(core_axis_name="c", subcore_axis_name="s")

    @functools.partial(
        pl.kernel,
        out_type=jax.ShapeDtypeStruct((n, D), jnp.float32),
        mesh=mesh,
        scratch_types=[
            pltpu.VMEM((ch,), jnp.int32),
            pltpu.VMEM((ch, D), jnp.float32),
            pltpu.SemaphoreType.DMA,
        ],
        compiler_params=pltpu.CompilerParams(use_tc_tiling_on_sc=False),
    )
    def k(tbl, ih, oh, idx_v, rows_v, sem):
        wid = lax.axis_index("s") * 2 + lax.axis_index("c")

        def body(c, _):
            off = wid * per + c * ch
            pltpu.sync_copy(ih.at[pl.ds(off, ch)], idx_v)
            pltpu.async_copy(tbl.at[idx_v], rows_v, sem).wait()
            pltpu.sync_copy(rows_v, oh.at[pl.ds(off, ch)])
            return 0

        lax.fori_loop(0, per // ch, body, 0)

    return k(table, idx)


def _dense_body(gf_ref, fnb_ref, sW1_ref, sb1_ref, sW2_ref, sb2_ref,
                wb_ref, out_ref):
    _, K, RT, C = gf_ref.shape
    OUT = wb_ref.shape[1] // _MM
    gflat = gf_ref[0].reshape(K * RT, C)
    h = jnp.maximum(
        jnp.dot(gflat, sW1_ref[:], preferred_element_type=jnp.float32)
        + sb1_ref[:], 0.0)
    zz = jnp.dot(h, sW2_ref[:], preferred_element_type=jnp.float32) + sb2_ref[:]
    zm = jnp.max(zz, axis=-1, keepdims=True)
    e = jnp.exp(zz - zm)
    s = e / jnp.sum(e, axis=-1, keepdims=True)
    s3 = s.reshape(K, RT, _MM)
    fnb3 = fnb_ref[0]
    acc = jnp.zeros((RT, OUT), jnp.float32)
    for m in range(_MM):
        gm = jnp.sum(fnb3 * s3[:, :, m:m + 1], axis=0)
        acc = acc + jnp.dot(gm, wb_ref[:, m * OUT:(m + 1) * OUT],
                            preferred_element_type=jnp.float32)
    out_ref[0] = acc


def _dense_call(gf, fnb, sW1, sb1r, sW2, sb2r, weightbank, RT=256):
    B, K, M, C = gf.shape
    OUT = weightbank.shape[1] // _MM
    grid = (B, M // RT)
    return pl.pallas_call(
        _dense_body,
        grid=grid,
        in_specs=[
            pl.BlockSpec((1, K, RT, C), lambda b, t: (b, 0, t, 0)),
            pl.BlockSpec((1, K, RT, C), lambda b, t: (b, 0, t, 0)),
            pl.BlockSpec(sW1.shape, lambda b, t: (0, 0)),
            pl.BlockSpec(sb1r.shape, lambda b, t: (0, 0)),
            pl.BlockSpec(sW2.shape, lambda b, t: (0, 0)),
            pl.BlockSpec(sb2r.shape, lambda b, t: (0, 0)),
            pl.BlockSpec(weightbank.shape, lambda b, t: (0, 0)),
        ],
        out_specs=pl.BlockSpec((1, RT, OUT), lambda b, t: (b, t, 0)),
        out_shape=jax.ShapeDtypeStruct((B, M, OUT), jnp.float32),
    )(gf, fnb, sW1, sb1r, sW2, sb2r, weightbank)


def _bn_body(x_ref, g_ref, b_ref, out_ref):
    B, OUT, M = x_ref.shape
    x = x_ref[:]
    cnt = jnp.float32(B * M)
    s0 = jnp.sum(x, axis=0)
    mean = (jnp.sum(s0, axis=1, keepdims=True) / cnt)
    diff = x - mean[None]
    d0 = jnp.sum(diff * diff, axis=0)
    var = jnp.sum(d0, axis=1, keepdims=True) / cnt
    xn = diff / jnp.sqrt(var + 1e-5)[None]
    y = g_ref[:] * xn + b_ref[:]
    out_ref[:] = jnp.maximum(y, 0.0)


def _bn_call(feats_t, gamma3, beta3):
    return pl.pallas_call(
        _bn_body,
        out_shape=jax.ShapeDtypeStruct(feats_t.shape, jnp.float32),
    )(feats_t, gamma3, beta3)


def kernel(xyz, feature, weightbank, sW1, sb1, sW2, sb2, gamma, beta):
    B, N, _ = xyz.shape
    C = feature.shape[1]
    M = N // 4
    OUT = weightbank.shape[1] // _MM

    xyz_t = xyz.transpose(0, 2, 1).reshape(B, 3, 8, N // 8)
    featT_flat = feature.transpose(0, 2, 1).reshape(B * N, C)

    fidx, px, py, pz = _fps_call(xyz_t)
    p2 = jnp.stack([px, py, pz], axis=-1)

    featM_flat = _sc_gather1(featT_flat, fidx.reshape(B * M))
    featM = featM_flat.reshape(B, M, C)
    featM_T = featM.transpose(0, 2, 1)

    kg, km = _knn_call(featM, featM_T)
    gf_flat, fnb_flat = _sc_gather2(featT_flat, kg.reshape(-1),
                                    featM_flat, km.reshape(-1))
    gf = gf_flat.reshape(B, _K, M, C)
    fnb = fnb_flat.reshape(B, _K, M, C)

    feats = _dense_call(gf, fnb, sW1, sb1.reshape(1, -1), sW2,
                        sb2.reshape(1, -1), weightbank)
    feats_t = feats.transpose(0, 2, 1)
    out = _bn_call(feats_t, gamma.reshape(1, OUT, 1),
                   beta.reshape(1, OUT, 1))
    return (p2, out)


# speedup vs baseline: 1.1834x; 1.1834x over previous
import functools

import jax
import jax.numpy as jnp
from jax import lax
from jax.experimental import pallas as pl
from jax.experimental.pallas import tpu as pltpu
from jax.experimental.pallas import tpu_sc as plsc

_K = 16
_MM = 8


def _fps_body(xyz_ref, fidx_ref, px_ref, py_ref, pz_ref):
    B, _, S, L = xyz_ref.shape
    N = S * L
    M = N // 4
    x = xyz_ref[:, 0]
    y = xyz_ref[:, 1]
    z = xyz_ref[:, 2]
    iota3 = (lax.broadcasted_iota(jnp.int32, (B, S, L), 1) * L
             + lax.broadcasted_iota(jnp.int32, (B, S, L), 2))
    boff = lax.broadcasted_iota(jnp.int32, (B, 1), 0) * N
    iota128 = lax.broadcasted_iota(jnp.int32, (B, 128), 1)

    def inner(j, st):
        dist, far, cx, cy, cz, pf, ppx, ppy, ppz = st
        lane = iota128 == j
        pf = jnp.where(lane, far[:, 0, :] + boff, pf)
        ppx = jnp.where(lane, cx[:, 0, :], ppx)
        ppy = jnp.where(lane, cy[:, 0, :], ppy)
        ppz = jnp.where(lane, cz[:, 0, :], ppz)
        dx = x - cx
        dy = y - cy
        dz = z - cz
        d = (dx * dx + dy * dy) + dz * dz
        dist = jnp.minimum(dist, d)
        v, ii, xx, yy, zz = dist, iota3, x, y, z
        w = L
        while w > 128:
            h = w // 2
            v1, v2 = v[:, :, :h], v[:, :, h:w]
            i1, i2 = ii[:, :, :h], ii[:, :, h:w]
            cond = (v1 > v2) | ((v1 == v2) & (i1 < i2))
            v = jnp.where(cond, v1, v2)
            ii = jnp.where(cond, i1, i2)
            xx = jnp.where(cond, xx[:, :, :h], xx[:, :, h:w])
            yy = jnp.where(cond, yy[:, :, :h], yy[:, :, h:w])
            zz = jnp.where(cond, zz[:, :, :h], zz[:, :, h:w])
            w = h
        mxs = jnp.max(jnp.max(v, axis=2, keepdims=True), axis=1,
                      keepdims=True)
        cand = jnp.where(v == mxs, ii, N)
        far2 = jnp.min(jnp.min(cand, axis=2, keepdims=True), axis=1,
                       keepdims=True)
        sel = ii == far2
        cx2 = jnp.sum(jnp.sum(jnp.where(sel, xx, 0.0), axis=2,
                              keepdims=True), axis=1, keepdims=True)
        cy2 = jnp.sum(jnp.sum(jnp.where(sel, yy, 0.0), axis=2,
                              keepdims=True), axis=1, keepdims=True)
        cz2 = jnp.sum(jnp.sum(jnp.where(sel, zz, 0.0), axis=2,
                              keepdims=True), axis=1, keepdims=True)
        return dist, far2, cx2, cy2, cz2, pf, ppx, ppy, ppz

    def outer(o, carry):
        dist, far, cx, cy, cz = carry
        pf0 = jnp.zeros((B, 128), jnp.int32)
        pp0 = jnp.zeros((B, 128), jnp.float32)
        st = lax.fori_loop(0, 128, inner,
                           (dist, far, cx, cy, cz, pf0, pp0, pp0, pp0))
        dist, far, cx, cy, cz, pf, ppx, ppy, ppz = st
        off = pl.multiple_of(o * 128, 128)
        fidx_ref[:, pl.ds(off, 128)] = pf
        px_ref[:, pl.ds(off, 128)] = ppx
        py_ref[:, pl.ds(off, 128)] = ppy
        pz_ref[:, pl.ds(off, 128)] = ppz
        return dist, far, cx, cy, cz

    dist0 = jnp.full((B, S, L), 1e10, dtype=jnp.float32)
    far0 = jnp.zeros((B, 1, 1), jnp.int32)
    lax.fori_loop(0, M // 128, outer,
                  (dist0, far0, x[:, 0:1, 0:1], y[:, 0:1, 0:1],
                   z[:, 0:1, 0:1]))


def _fps_call(xyz_t):
    B, _, S, L = xyz_t.shape
    M = S * L // 4
    return pl.pallas_call(
        _fps_body,
        out_shape=[
            jax.ShapeDtypeStruct((B, M), jnp.int32),
            jax.ShapeDtypeStruct((B, M), jnp.float32),
            jax.ShapeDtypeStruct((B, M), jnp.float32),
            jax.ShapeDtypeStruct((B, M), jnp.float32),
        ],
    )(xyz_t)


def _knn_body(fm_ref, fmt_ref, kg_ref, km_ref):
    _, RT, C = fm_ref.shape
    _, _, M = fmt_ref.shape
    N = M * 4
    t = fm_ref[0]
    Ft = fmt_ref[0]
    ip = jnp.dot(t, Ft, preferred_element_type=jnp.float32)
    inner = -2.0 * ip
    xxF = jnp.sum(Ft * Ft, axis=0, keepdims=True)
    xxT = jnp.sum(t * t, axis=1, keepdims=True)
    pd = (-xxF - inner) - xxT
    iota2 = lax.broadcasted_iota(jnp.int32, (RT, M), 1)
    b = pl.program_id(0)
    neg = jnp.float32(-jnp.inf)
    for k in range(_K):
        mx = jnp.max(pd, axis=1, keepdims=True)
        idx = jnp.min(jnp.where(pd == mx, iota2, M), axis=1)
        kg_ref[0, k, :] = idx + b * N
        km_ref[0, k, :] = idx + b * M
        pd = jnp.where(iota2 == idx[:, None], neg, pd)


def _knn_call(featM, featM_T, RT=512):
    B, M, C = featM.shape
    grid = (B, M // RT)
    return pl.pallas_call(
        _knn_body,
        grid=grid,
        in_specs=[
            pl.BlockSpec((1, RT, C), lambda b, t: (b, t, 0)),
            pl.BlockSpec((1, C, M), lambda b, t: (b, 0, 0)),
        ],
        out_specs=[
            pl.BlockSpec((1, _K, RT), lambda b, t: (b, 0, t)),
            pl.BlockSpec((1, _K, RT), lambda b, t: (b, 0, t)),
        ],
        out_shape=[
            jax.ShapeDtypeStruct((B, _K, M), jnp.int32),
            jax.ShapeDtypeStruct((B, _K, M), jnp.int32),
        ],
    )(featM, featM_T)


def _sc_gather2(tableA, idxA, tableB, idxB, ch=128):
    D = tableA.shape[1]
    nA = idxA.shape[0]
    nB = idxB.shape[0]
    nw = 32
    perA = nA // nw
    perB = nB // nw
    mesh = plsc.VectorSubcoreMesh(core_axis_name="c", subcore_axis_name="s")

    @functools.partial(
        pl.kernel,
        out_type=[
            jax.ShapeDtypeStruct((nA, D), jnp.float32),
            jax.ShapeDtypeStruct((nB, D), jnp.float32),
        ],
        mesh=mesh,
        scratch_types=[
            pltpu.VMEM((ch,), jnp.int32),
            pltpu.VMEM((ch, D), jnp.float32),
            pltpu.SemaphoreType.DMA,
        ],
        compiler_params=pltpu.CompilerParams(use_tc_tiling_on_sc=False),
    )
    def k(tA, iA, tB, iB, oA, oB, idx_v, rows_v, sem):
        wid = lax.axis_index("s") * 2 + lax.axis_index("c")

        def one(c, tbl, ih, oh, base):
            off = base + c * ch
            pltpu.sync_copy(ih.at[pl.ds(off, ch)], idx_v)
            pltpu.async_copy(tbl.at[idx_v], rows_v, sem).wait()
            pltpu.sync_copy(rows_v, oh.at[pl.ds(off, ch)])

        def bodyA(c, _):
            one(c, tA, iA, oA, wid * perA)
            return 0

        def bodyB(c, _):
            one(c, tB, iB, oB, wid * perB)
            return 0

        lax.fori_loop(0, perA // ch, bodyA, 0)
        lax.fori_loop(0, perB // ch, bodyB, 0)

    return k(tableA, idxA, tableB, idxB)


def _sc_gather1(table, idx, ch=128):
    D = table.shape[1]
    n = idx.shape[0]
    nw = 32
    per = n // nw
    mesh = plsc.VectorSubcoreMesh(core_axis_name="c", subcore_axis_name="s")

    @functools.partial(
        pl.kernel,
        out_type=jax.ShapeDtypeStruct((n, D), jnp.float32),
        mesh=mesh,
        scratch_types=[
            pltpu.VMEM((ch,), jnp.int32),
            pltpu.VMEM((ch, D), jnp.float32),
            pltpu.SemaphoreType.DMA,
        ],
        compiler_params=pltpu.CompilerParams(use_tc_tiling_on_sc=False),
    )
    def k(tbl, ih, oh, idx_v, rows_v, sem):
        wid = lax.axis_index("s") * 2 + lax.axis_index("c")

        def body(c, _):
            off = wid * per + c * ch
            pltpu.sync_copy(ih.at[pl.ds(off, ch)], idx_v)
            pltpu.async_copy(tbl.at[idx_v], rows_v, sem).wait()
            pltpu.sync_copy(rows_v, oh.at[pl.ds(off, ch)])
            return 0

        lax.fori_loop(0, per // ch, body, 0)

    return k(table, idx)


def _dense_body(gf_ref, fnb_ref, sW1_ref, sb1_ref, sW2_ref, sb2_ref,
                wb_ref, out_ref):
    _, K, RT, C = gf_ref.shape
    OUT = wb_ref.shape[1] // _MM
    gflat = gf_ref[0].reshape(K * RT, C)
    h = jnp.maximum(
        jnp.dot(gflat, sW1_ref[:], preferred_element_type=jnp.float32)
        + sb1_ref[:], 0.0)
    zz = jnp.dot(h, sW2_ref[:], preferred_element_type=jnp.float32) + sb2_ref[:]
    zm = jnp.max(zz, axis=-1, keepdims=True)
    e = jnp.exp(zz - zm)
    s = e / jnp.sum(e, axis=-1, keepdims=True)
    s3 = s.reshape(K, RT, _MM)
    fnb3 = fnb_ref[0]
    acc = jnp.zeros((OUT, RT), jnp.float32)
    for m in range(_MM):
        gm = jnp.sum(fnb3 * s3[:, :, m:m + 1], axis=0)
        acc = acc + lax.dot_general(
            wb_ref[:, m * OUT:(m + 1) * OUT], gm,
            (((0,), (1,)), ((), ())),
            preferred_element_type=jnp.float32)
    out_ref[0] = acc


def _dense_call(gf, fnb, sW1, sb1r, sW2, sb2r, weightbank, RT=256):
    B, K, M, C = gf.shape
    OUT = weightbank.shape[1] // _MM
    grid = (B, M // RT)
    return pl.pallas_call(
        _dense_body,
        grid=grid,
        in_specs=[
            pl.BlockSpec((1, K, RT, C), lambda b, t: (b, 0, t, 0)),
            pl.BlockSpec((1, K, RT, C), lambda b, t: (b, 0, t, 0)),
            pl.BlockSpec(sW1.shape, lambda b, t: (0, 0)),
            pl.BlockSpec(sb1r.shape, lambda b, t: (0, 0)),
            pl.BlockSpec(sW2.shape, lambda b, t: (0, 0)),
            pl.BlockSpec(sb2r.shape, lambda b, t: (0, 0)),
            pl.BlockSpec(weightbank.shape, lambda b, t: (0, 0)),
        ],
        out_specs=pl.BlockSpec((1, OUT, RT), lambda b, t: (b, 0, t)),
        out_shape=jax.ShapeDtypeStruct((B, OUT, M), jnp.float32),
    )(gf, fnb, sW1, sb1r, sW2, sb2r, weightbank)


def _bn_body(x_ref, g_ref, b_ref, out_ref):
    B, OUT, M = x_ref.shape
    x = x_ref[:]
    cnt = jnp.float32(B * M)
    s0 = jnp.sum(x, axis=0)
    mean = (jnp.sum(s0, axis=1, keepdims=True) / cnt)
    diff = x - mean[None]
    d0 = jnp.sum(diff * diff, axis=0)
    var = jnp.sum(d0, axis=1, keepdims=True) / cnt
    xn = diff / jnp.sqrt(var + 1e-5)[None]
    y = g_ref[:] * xn + b_ref[:]
    out_ref[:] = jnp.maximum(y, 0.0)


def _bn_call(feats_t, gamma3, beta3):
    return pl.pallas_call(
        _bn_body,
        out_shape=jax.ShapeDtypeStruct(feats_t.shape, jnp.float32),
    )(feats_t, gamma3, beta3)


def kernel(xyz, feature, weightbank, sW1, sb1, sW2, sb2, gamma, beta):
    B, N, _ = xyz.shape
    C = feature.shape[1]
    M = N // 4
    OUT = weightbank.shape[1] // _MM

    xyz_t = xyz.transpose(0, 2, 1).reshape(B, 3, 8, N // 8)
    featT_flat = feature.transpose(0, 2, 1).reshape(B * N, C)

    fidx, px, py, pz = _fps_call(xyz_t)
    p2 = jnp.stack([px, py, pz], axis=-1)

    featM_flat = _sc_gather1(featT_flat, fidx.reshape(B * M))
    featM = featM_flat.reshape(B, M, C)
    featM_T = featM.transpose(0, 2, 1)

    kg, km = _knn_call(featM, featM_T)
    gf_flat, fnb_flat = _sc_gather2(featT_flat, kg.reshape(-1),
                                    featM_flat, km.reshape(-1))
    gf = gf_flat.reshape(B, _K, M, C)
    fnb = fnb_flat.reshape(B, _K, M, C)

    feats_t = _dense_call(gf, fnb, sW1, sb1.reshape(1, -1), sW2,
                          sb2.reshape(1, -1), weightbank)
    out = _bn_call(feats_t, gamma.reshape(1, OUT, 1),
                   beta.reshape(1, OUT, 1))
    return (p2, out)


# speedup vs baseline: 1.1995x; 1.0136x over previous
import functools

import jax
import jax.numpy as jnp
from jax import lax
from jax.experimental import pallas as pl
from jax.experimental.pallas import tpu as pltpu
from jax.experimental.pallas import tpu_sc as plsc

_K = 16
_MM = 8


def _fps_body(xyz_ref, fidx_ref, px_ref, py_ref, pz_ref):
    B, _, S, L = xyz_ref.shape
    N = S * L
    M = N // 4
    x = xyz_ref[:, 0]
    y = xyz_ref[:, 1]
    z = xyz_ref[:, 2]
    iota3 = (lax.broadcasted_iota(jnp.int32, (B, S, L), 1) * L
             + lax.broadcasted_iota(jnp.int32, (B, S, L), 2))
    boff = lax.broadcasted_iota(jnp.int32, (B, 1), 0) * N
    iota128 = lax.broadcasted_iota(jnp.int32, (B, 128), 1)

    def inner(j, st):
        dist, far, cx, cy, cz, pf, ppx, ppy, ppz = st
        lane = iota128 == j
        pf = jnp.where(lane, far[:, 0, :] + boff, pf)
        ppx = jnp.where(lane, cx[:, 0, :], ppx)
        ppy = jnp.where(lane, cy[:, 0, :], ppy)
        ppz = jnp.where(lane, cz[:, 0, :], ppz)
        dx = x - cx
        dy = y - cy
        dz = z - cz
        d = (dx * dx + dy * dy) + dz * dz
        dist = jnp.minimum(dist, d)
        v, ii, xx, yy, zz = dist, iota3, x, y, z
        w = L
        while w > 128:
            h = w // 2
            v1, v2 = v[:, :, :h], v[:, :, h:w]
            i1, i2 = ii[:, :, :h], ii[:, :, h:w]
            cond = (v1 > v2) | ((v1 == v2) & (i1 < i2))
            v = jnp.where(cond, v1, v2)
            ii = jnp.where(cond, i1, i2)
            xx = jnp.where(cond, xx[:, :, :h], xx[:, :, h:w])
            yy = jnp.where(cond, yy[:, :, :h], yy[:, :, h:w])
            zz = jnp.where(cond, zz[:, :, :h], zz[:, :, h:w])
            w = h
        mxs = jnp.max(jnp.max(v, axis=2, keepdims=True), axis=1,
                      keepdims=True)
        cand = jnp.where(v == mxs, ii, N)
        far2 = jnp.min(jnp.min(cand, axis=2, keepdims=True), axis=1,
                       keepdims=True)
        sel = ii == far2
        cx2 = jnp.sum(jnp.sum(jnp.where(sel, xx, 0.0), axis=2,
                              keepdims=True), axis=1, keepdims=True)
        cy2 = jnp.sum(jnp.sum(jnp.where(sel, yy, 0.0), axis=2,
                              keepdims=True), axis=1, keepdims=True)
        cz2 = jnp.sum(jnp.sum(jnp.where(sel, zz, 0.0), axis=2,
                              keepdims=True), axis=1, keepdims=True)
        return dist, far2, cx2, cy2, cz2, pf, ppx, ppy, ppz

    def outer(o, carry):
        dist, far, cx, cy, cz = carry
        pf0 = jnp.zeros((B, 128), jnp.int32)
        pp0 = jnp.zeros((B, 128), jnp.float32)
        st = lax.fori_loop(0, 128, inner,
                           (dist, far, cx, cy, cz, pf0, pp0, pp0, pp0))
        dist, far, cx, cy, cz, pf, ppx, ppy, ppz = st
        off = pl.multiple_of(o * 128, 128)
        fidx_ref[:, pl.ds(off, 128)] = pf
        px_ref[:, pl.ds(off, 128)] = ppx
        py_ref[:, pl.ds(off, 128)] = ppy
        pz_ref[:, pl.ds(off, 128)] = ppz
        return dist, far, cx, cy, cz

    dist0 = jnp.full((B, S, L), 1e10, dtype=jnp.float32)
    far0 = jnp.zeros((B, 1, 1), jnp.int32)
    lax.fori_loop(0, M // 128, outer,
                  (dist0, far0, x[:, 0:1, 0:1], y[:, 0:1, 0:1],
                   z[:, 0:1, 0:1]))


def _fps_call(xyz_t):
    B, _, S, L = xyz_t.shape
    M = S * L // 4
    return pl.pallas_call(
        _fps_body,
        out_shape=[
            jax.ShapeDtypeStruct((B, M), jnp.int32),
            jax.ShapeDtypeStruct((B, M), jnp.float32),
            jax.ShapeDtypeStruct((B, M), jnp.float32),
            jax.ShapeDtypeStruct((B, M), jnp.float32),
        ],
    )(xyz_t)


def _knn_body(fm_ref, fmt_ref, kg_ref, km_ref):
    _, RT, C = fm_ref.shape
    _, _, M = fmt_ref.shape
    N = M * 4
    t = fm_ref[0]
    Ft = fmt_ref[0]
    ip = jnp.dot(t, Ft, preferred_element_type=jnp.float32)
    inner = -2.0 * ip
    xxF = jnp.sum(Ft * Ft, axis=0, keepdims=True)
    xxT = jnp.sum(t * t, axis=1, keepdims=True)
    pd = (-xxF - inner) - xxT
    iota2 = lax.broadcasted_iota(jnp.int32, (RT, M), 1)
    b = pl.program_id(0)
    neg = jnp.float32(-jnp.inf)
    for k in range(_K):
        mx = jnp.max(pd, axis=1, keepdims=True)
        idx = jnp.min(jnp.where(pd == mx, iota2, M), axis=1)
        kg_ref[0, k, :] = idx + b * N
        km_ref[0, k, :] = idx + b * M
        pd = jnp.where(iota2 == idx[:, None], neg, pd)


def _knn_call(featM, featM_T, RT=512):
    B, M, C = featM.shape
    grid = (B, M // RT)
    return pl.pallas_call(
        _knn_body,
        grid=grid,
        in_specs=[
            pl.BlockSpec((1, RT, C), lambda b, t: (b, t, 0)),
            pl.BlockSpec((1, C, M), lambda b, t: (b, 0, 0)),
        ],
        out_specs=[
            pl.BlockSpec((1, _K, RT), lambda b, t: (b, 0, t)),
            pl.BlockSpec((1, _K, RT), lambda b, t: (b, 0, t)),
        ],
        out_shape=[
            jax.ShapeDtypeStruct((B, _K, M), jnp.int32),
            jax.ShapeDtypeStruct((B, _K, M), jnp.int32),
        ],
    )(featM, featM_T)


def _sc_gather2(tableA, idxA, tableB, idxB, ch=128):
    D = tableA.shape[1]
    nA = idxA.shape[0]
    nB = idxB.shape[0]
    nw = 32
    perA = nA // nw
    perB = nB // nw
    mesh = plsc.VectorSubcoreMesh(core_axis_name="c", subcore_axis_name="s")

    nf = 8
    blk = nf * ch

    @functools.partial(
        pl.kernel,
        out_type=[
            jax.ShapeDtypeStruct((nA, D), jnp.float32),
            jax.ShapeDtypeStruct((nB, D), jnp.float32),
        ],
        mesh=mesh,
        scratch_types=[
            pltpu.VMEM((perA,), jnp.int32),
            pltpu.VMEM((perB,), jnp.int32),
            pltpu.VMEM((blk, D), jnp.float32),
            pltpu.SemaphoreType.DMA,
        ],
        compiler_params=pltpu.CompilerParams(use_tc_tiling_on_sc=False),
    )
    def k(tA, iA, tB, iB, oA, oB, idxa_v, idxb_v, rows_v, sem):
        wid = lax.axis_index("s") * 2 + lax.axis_index("c")
        pltpu.sync_copy(iA.at[pl.ds(wid * perA, perA)], idxa_v)
        pltpu.sync_copy(iB.at[pl.ds(wid * perB, perB)], idxb_v)

        def mk(tbl, idx_v, oh, base, per):
            def round_(r, _):
                off = r * blk
                cps = []
                for s in range(nf):
                    cps.append(pltpu.async_copy(
                        tbl.at[idx_v.at[pl.ds(off + s * ch, ch)]],
                        rows_v.at[pl.ds(s * ch, ch)], sem))
                for cp in cps:
                    cp.wait()
                pltpu.sync_copy(rows_v, oh.at[pl.ds(base + off, blk)])
                return 0
            lax.fori_loop(0, per // blk, round_, 0)

        mk(tA, idxa_v, oA, wid * perA, perA)
        mk(tB, idxb_v, oB, wid * perB, perB)

    return k(tableA, idxA, tableB, idxB)


def _sc_gather1(table, idx, ch=128):
    D = table.shape[1]
    n = idx.shape[0]
    nw = 32
    per = n // nw
    mesh = plsc.VectorSubcoreMesh(core_axis_name="c", subcore_axis_name="s")

    @functools.partial(
        pl.kernel,
        out_type=jax.ShapeDtypeStruct((n, D), jnp.float32),
        mesh=mesh,
        scratch_types=[
            pltpu.VMEM((ch,), jnp.int32),
            pltpu.VMEM((ch, D), jnp.float32),
            pltpu.SemaphoreType.DMA,
        ],
        compiler_params=pltpu.CompilerParams(use_tc_tiling_on_sc=False),
    )
    def k(tbl, ih, oh, idx_v, rows_v, sem):
        wid = lax.axis_index("s") * 2 + lax.axis_index("c")

        def body(c, _):
            off = wid * per + c * ch
            pltpu.sync_copy(ih.at[pl.ds(off, ch)], idx_v)
            pltpu.async_copy(tbl.at[idx_v], rows_v, sem).wait()
            pltpu.sync_copy(rows_v, oh.at[pl.ds(off, ch)])
            return 0

        lax.fori_loop(0, per // ch, body, 0)

    return k(table, idx)


def _dense_body(gf_ref, fnb_ref, sW1_ref, sb1_ref, sW2_ref, sb2_ref,
                wb_ref, out_ref):
    _, K, RT, C = gf_ref.shape
    OUT = wb_ref.shape[1] // _MM
    gflat = gf_ref[0].reshape(K * RT, C)
    h = jnp.maximum(
        jnp.dot(gflat, sW1_ref[:], preferred_element_type=jnp.float32)
        + sb1_ref[:], 0.0)
    zz = jnp.dot(h, sW2_ref[:], preferred_element_type=jnp.float32) + sb2_ref[:]
    zm = jnp.max(zz, axis=-1, keepdims=True)
    e = jnp.exp(zz - zm)
    s = e / jnp.sum(e, axis=-1, keepdims=True)
    s3 = s.reshape(K, RT, _MM)
    fnb3 = fnb_ref[0]
    acc = jnp.zeros((OUT, RT), jnp.float32)
    for m in range(_MM):
        gm = jnp.sum(fnb3 * s3[:, :, m:m + 1], axis=0)
        acc = acc + lax.dot_general(
            wb_ref[:, m * OUT:(m + 1) * OUT], gm,
            (((0,), (1,)), ((), ())),
            preferred_element_type=jnp.float32)
    out_ref[0] = acc


def _dense_call(gf, fnb, sW1, sb1r, sW2, sb2r, weightbank, RT=256):
    B, K, M, C = gf.shape
    OUT = weightbank.shape[1] // _MM
    grid = (B, M // RT)
    return pl.pallas_call(
        _dense_body,
        grid=grid,
        in_specs=[
            pl.BlockSpec((1, K, RT, C), lambda b, t: (b, 0, t, 0)),
            pl.BlockSpec((1, K, RT, C), lambda b, t: (b, 0, t, 0)),
            pl.BlockSpec(sW1.shape, lambda b, t: (0, 0)),
            pl.BlockSpec(sb1r.shape, lambda b, t: (0, 0)),
            pl.BlockSpec(sW2.shape, lambda b, t: (0, 0)),
            pl.BlockSpec(sb2r.shape, lambda b, t: (0, 0)),
            pl.BlockSpec(weightbank.shape, lambda b, t: (0, 0)),
        ],
        out_specs=pl.BlockSpec((1, OUT, RT), lambda b, t: (b, 0, t)),
        out_shape=jax.ShapeDtypeStruct((B, OUT, M), jnp.float32),
    )(gf, fnb, sW1, sb1r, sW2, sb2r, weightbank)


def _bn_body(x_ref, g_ref, b_ref, out_ref):
    B, OUT, M = x_ref.shape
    x = x_ref[:]
    cnt = jnp.float32(B * M)
    s0 = jnp.sum(x, axis=0)
    mean = (jnp.sum(s0, axis=1, keepdims=True) / cnt)
    diff = x - mean[None]
    d0 = jnp.sum(diff * diff, axis=0)
    var = jnp.sum(d0, axis=1, keepdims=True) / cnt
    xn = diff / jnp.sqrt(var + 1e-5)[None]
    y = g_ref[:] * xn + b_ref[:]
    out_ref[:] = jnp.maximum(y, 0.0)


def _bn_call(feats_t, gamma3, beta3):
    return pl.pallas_call(
        _bn_body,
        out_shape=jax.ShapeDtypeStruct(feats_t.shape, jnp.float32),
    )(feats_t, gamma3, beta3)


def kernel(xyz, feature, weightbank, sW1, sb1, sW2, sb2, gamma, beta):
    B, N, _ = xyz.shape
    C = feature.shape[1]
    M = N // 4
    OUT = weightbank.shape[1] // _MM

    xyz_t = xyz.transpose(0, 2, 1).reshape(B, 3, 8, N // 8)
    featT_flat = feature.transpose(0, 2, 1).reshape(B * N, C)

    fidx, px, py, pz = _fps_call(xyz_t)
    p2 = jnp.stack([px, py, pz], axis=-1)

    featM_flat = _sc_gather1(featT_flat, fidx.reshape(B * M))
    featM = featM_flat.reshape(B, M, C)
    featM_T = featM.transpose(0, 2, 1)

    kg, km = _knn_call(featM, featM_T)
    gf_flat, fnb_flat = _sc_gather2(featT_flat, kg.reshape(-1),
                                    featM_flat, km.reshape(-1))
    gf = gf_flat.reshape(B, _K, M, C)
    fnb = fnb_flat.reshape(B, _K, M, C)

    feats_t = _dense_call(gf, fnb, sW1, sb1.reshape(1, -1), sW2,
                          sb2.reshape(1, -1), weightbank)
    out = _bn_call(feats_t, gamma.reshape(1, OUT, 1),
                   beta.reshape(1, OUT, 1))
    return (p2, out)


# speedup vs baseline: 1.3229x; 1.1029x over previous
import functools

import jax
import jax.numpy as jnp
from jax import lax
from jax.experimental import pallas as pl
from jax.experimental.pallas import tpu as pltpu
from jax.experimental.pallas import tpu_sc as plsc

_K = 16
_MM = 8


def _fps_body(xyz_ref, fidx_ref, px_ref, py_ref, pz_ref):
    B, _, S, L = xyz_ref.shape
    N = S * L
    M = N // 4
    x = xyz_ref[:, 0]
    y = xyz_ref[:, 1]
    z = xyz_ref[:, 2]
    iota3 = (lax.broadcasted_iota(jnp.int32, (B, S, L), 1) * L
             + lax.broadcasted_iota(jnp.int32, (B, S, L), 2))
    boff = lax.broadcasted_iota(jnp.int32, (B, 1), 0) * N
    iota128 = lax.broadcasted_iota(jnp.int32, (B, 128), 1)

    def inner(j, st):
        dist, far, cx, cy, cz, pf, ppx, ppy, ppz = st
        lane = iota128 == j
        pf = jnp.where(lane, far[:, 0, :] + boff, pf)
        ppx = jnp.where(lane, cx[:, 0, :], ppx)
        ppy = jnp.where(lane, cy[:, 0, :], ppy)
        ppz = jnp.where(lane, cz[:, 0, :], ppz)
        dx = x - cx
        dy = y - cy
        dz = z - cz
        d = (dx * dx + dy * dy) + dz * dz
        dist = jnp.minimum(dist, d)
        v, ii, xx, yy, zz = dist, iota3, x, y, z
        w = L
        while w > 128:
            h = w // 2
            v1, v2 = v[:, :, :h], v[:, :, h:w]
            i1, i2 = ii[:, :, :h], ii[:, :, h:w]
            cond = (v1 > v2) | ((v1 == v2) & (i1 < i2))
            v = jnp.where(cond, v1, v2)
            ii = jnp.where(cond, i1, i2)
            xx = jnp.where(cond, xx[:, :, :h], xx[:, :, h:w])
            yy = jnp.where(cond, yy[:, :, :h], yy[:, :, h:w])
            zz = jnp.where(cond, zz[:, :, :h], zz[:, :, h:w])
            w = h
        mxs = jnp.max(jnp.max(v, axis=2, keepdims=True), axis=1,
                      keepdims=True)
        cand = jnp.where(v == mxs, ii, N)
        far2 = jnp.min(jnp.min(cand, axis=2, keepdims=True), axis=1,
                       keepdims=True)
        sel = ii == far2
        cx2 = jnp.sum(jnp.sum(jnp.where(sel, xx, 0.0), axis=2,
                              keepdims=True), axis=1, keepdims=True)
        cy2 = jnp.sum(jnp.sum(jnp.where(sel, yy, 0.0), axis=2,
                              keepdims=True), axis=1, keepdims=True)
        cz2 = jnp.sum(jnp.sum(jnp.where(sel, zz, 0.0), axis=2,
                              keepdims=True), axis=1, keepdims=True)
        return dist, far2, cx2, cy2, cz2, pf, ppx, ppy, ppz

    def outer(o, carry):
        dist, far, cx, cy, cz = carry
        pf0 = jnp.zeros((B, 128), jnp.int32)
        pp0 = jnp.zeros((B, 128), jnp.float32)
        def inner2(j2, st2):
            return inner(2 * j2 + 1, inner(2 * j2, st2))

        st = lax.fori_loop(0, 64, inner2,
                           (dist, far, cx, cy, cz, pf0, pp0, pp0, pp0))
        dist, far, cx, cy, cz, pf, ppx, ppy, ppz = st
        off = pl.multiple_of(o * 128, 128)
        fidx_ref[:, pl.ds(off, 128)] = pf
        px_ref[:, pl.ds(off, 128)] = ppx
        py_ref[:, pl.ds(off, 128)] = ppy
        pz_ref[:, pl.ds(off, 128)] = ppz
        return dist, far, cx, cy, cz

    dist0 = jnp.full((B, S, L), 1e10, dtype=jnp.float32)
    far0 = jnp.zeros((B, 1, 1), jnp.int32)
    lax.fori_loop(0, M // 128, outer,
                  (dist0, far0, x[:, 0:1, 0:1], y[:, 0:1, 0:1],
                   z[:, 0:1, 0:1]))


def _fps_call(xyz_t):
    B, _, S, L = xyz_t.shape
    M = S * L // 4
    return pl.pallas_call(
        _fps_body,
        out_shape=[
            jax.ShapeDtypeStruct((B, M), jnp.int32),
            jax.ShapeDtypeStruct((B, M), jnp.float32),
            jax.ShapeDtypeStruct((B, M), jnp.float32),
            jax.ShapeDtypeStruct((B, M), jnp.float32),
        ],
    )(xyz_t)


def _knn_body(fm_ref, fmt_ref, kg_ref, km_ref):
    _, RT, C = fm_ref.shape
    _, _, M = fmt_ref.shape
    N = M * 4
    t = fm_ref[0]
    Ft = fmt_ref[0]
    ip = jnp.dot(t, Ft, preferred_element_type=jnp.float32)
    inner = -2.0 * ip
    xxF = jnp.sum(Ft * Ft, axis=0, keepdims=True)
    xxT = jnp.sum(t * t, axis=1, keepdims=True)
    pd = (-xxF - inner) - xxT
    iota2 = lax.broadcasted_iota(jnp.int32, (RT, M), 1)
    b = pl.program_id(0)
    neg = jnp.float32(-jnp.inf)
    for k in range(_K):
        mx = jnp.max(pd, axis=1, keepdims=True)
        idx = jnp.min(jnp.where(pd == mx, iota2, M), axis=1)
        kg_ref[0, k, :] = idx + b * N
        km_ref[0, k, :] = idx + b * M
        pd = jnp.where(iota2 == idx[:, None], neg, pd)


def _knn_call(featM, featM_T, RT=512):
    B, M, C = featM.shape
    grid = (B, M // RT)
    return pl.pallas_call(
        _knn_body,
        grid=grid,
        in_specs=[
            pl.BlockSpec((1, RT, C), lambda b, t: (b, t, 0)),
            pl.BlockSpec((1, C, M), lambda b, t: (b, 0, 0)),
        ],
        out_specs=[
            pl.BlockSpec((1, _K, RT), lambda b, t: (b, 0, t)),
            pl.BlockSpec((1, _K, RT), lambda b, t: (b, 0, t)),
        ],
        out_shape=[
            jax.ShapeDtypeStruct((B, _K, M), jnp.int32),
            jax.ShapeDtypeStruct((B, _K, M), jnp.int32),
        ],
    )(featM, featM_T)


def _sc_gather2(tableA, idxA, tableB, idxB, ch=128):
    D = tableA.shape[1]
    nA = idxA.shape[0]
    nB = idxB.shape[0]
    nw = 32
    perA = nA // nw
    perB = nB // nw
    mesh = plsc.VectorSubcoreMesh(core_axis_name="c", subcore_axis_name="s")

    nf = 8
    blk = nf * ch

    @functools.partial(
        pl.kernel,
        out_type=[
            jax.ShapeDtypeStruct((nA, D), jnp.float32),
            jax.ShapeDtypeStruct((nB, D), jnp.float32),
        ],
        mesh=mesh,
        scratch_types=[
            pltpu.VMEM((perA,), jnp.int32),
            pltpu.VMEM((perB,), jnp.int32),
            pltpu.VMEM((blk, D), jnp.float32),
            pltpu.SemaphoreType.DMA,
        ],
        compiler_params=pltpu.CompilerParams(use_tc_tiling_on_sc=False),
    )
    def k(tA, iA, tB, iB, oA, oB, idxa_v, idxb_v, rows_v, sem):
        wid = lax.axis_index("s") * 2 + lax.axis_index("c")
        pltpu.sync_copy(iA.at[pl.ds(wid * perA, perA)], idxa_v)
        pltpu.sync_copy(iB.at[pl.ds(wid * perB, perB)], idxb_v)

        def mk(tbl, idx_v, oh, base, per):
            def round_(r, _):
                off = r * blk
                cps = []
                for s in range(nf):
                    cps.append(pltpu.async_copy(
                        tbl.at[idx_v.at[pl.ds(off + s * ch, ch)]],
                        rows_v.at[pl.ds(s * ch, ch)], sem))
                for cp in cps:
                    cp.wait()
                pltpu.sync_copy(rows_v, oh.at[pl.ds(base + off, blk)])
                return 0
            lax.fori_loop(0, per // blk, round_, 0)

        mk(tA, idxa_v, oA, wid * perA, perA)
        mk(tB, idxb_v, oB, wid * perB, perB)

    return k(tableA, idxA, tableB, idxB)


def _sc_gather1(table, idx, ch=128):
    D = table.shape[1]
    n = idx.shape[0]
    nw = 32
    per = n // nw
    mesh = plsc.VectorSubcoreMesh(core_axis_name="c", subcore_axis_name="s")

    @functools.partial(
        pl.kernel,
        out_type=jax.ShapeDtypeStruct((n, D), jnp.float32),
        mesh=mesh,
        scratch_types=[
            pltpu.VMEM((ch,), jnp.int32),
            pltpu.VMEM((ch, D), jnp.float32),
            pltpu.SemaphoreType.DMA,
        ],
        compiler_params=pltpu.CompilerParams(use_tc_tiling_on_sc=False),
    )
    def k(tbl, ih, oh, idx_v, rows_v, sem):
        wid = lax.axis_index("s") * 2 + lax.axis_index("c")

        def body(c, _):
            off = wid * per + c * ch
            pltpu.sync_copy(ih.at[pl.ds(off, ch)], idx_v)
            pltpu.async_copy(tbl.at[idx_v], rows_v, sem).wait()
            pltpu.sync_copy(rows_v, oh.at[pl.ds(off, ch)])
            return 0

        lax.fori_loop(0, per // ch, body, 0)

    return k(table, idx)


def _dense_body(gf_ref, fnb_ref, sW1_ref, sb1_ref, sW2_ref, sb2_ref,
                wb_ref, out_ref):
    _, K, RT, C = gf_ref.shape
    OUT = wb_ref.shape[1] // _MM
    gflat = gf_ref[0].reshape(K * RT, C)
    h = jnp.maximum(
        jnp.dot(gflat, sW1_ref[:], preferred_element_type=jnp.float32)
        + sb1_ref[:], 0.0)
    zz = jnp.dot(h, sW2_ref[:], preferred_element_type=jnp.float32) + sb2_ref[:]
    zm = jnp.max(zz, axis=-1, keepdims=True)
    e = jnp.exp(zz - zm)
    s = e / jnp.sum(e, axis=-1, keepdims=True)
    s3 = s.reshape(K, RT, _MM)
    fnb3 = fnb_ref[0]
    acc = jnp.zeros((OUT, RT), jnp.float32)
    for m in range(_MM):
        gm = jnp.sum(fnb3 * s3[:, :, m:m + 1], axis=0)
        acc = acc + lax.dot_general(
            wb_ref[:, m * OUT:(m + 1) * OUT], gm,
            (((0,), (1,)), ((), ())),
            preferred_element_type=jnp.float32)
    out_ref[0] = acc


def _dense_call(gf, fnb, sW1, sb1r, sW2, sb2r, weightbank, RT=256):
    B, K, M, C = gf.shape
    OUT = weightbank.shape[1] // _MM
    grid = (B, M // RT)
    return pl.pallas_call(
        _dense_body,
        grid=grid,
        in_specs=[
            pl.BlockSpec((1, K, RT, C), lambda b, t: (b, 0, t, 0)),
            pl.BlockSpec((1, K, RT, C), lambda b, t: (b, 0, t, 0)),
            pl.BlockSpec(sW1.shape, lambda b, t: (0, 0)),
            pl.BlockSpec(sb1r.shape, lambda b, t: (0, 0)),
            pl.BlockSpec(sW2.shape, lambda b, t: (0, 0)),
            pl.BlockSpec(sb2r.shape, lambda b, t: (0, 0)),
            pl.BlockSpec(weightbank.shape, lambda b, t: (0, 0)),
        ],
        out_specs=pl.BlockSpec((1, OUT, RT), lambda b, t: (b, 0, t)),
        out_shape=jax.ShapeDtypeStruct((B, OUT, M), jnp.float32),
    )(gf, fnb, sW1, sb1r, sW2, sb2r, weightbank)


def _bn_body(x_ref, g_ref, b_ref, out_ref):
    B, OUT, M = x_ref.shape
    x = x_ref[:]
    cnt = jnp.float32(B * M)
    s0 = jnp.sum(x, axis=0)
    mean = (jnp.sum(s0, axis=1, keepdims=True) / cnt)
    diff = x - mean[None]
    d0 = jnp.sum(diff * diff, axis=0)
    var = jnp.sum(d0, axis=1, keepdims=True) / cnt
    xn = diff / jnp.sqrt(var + 1e-5)[None]
    y = g_ref[:] * xn + b_ref[:]
    out_ref[:] = jnp.maximum(y, 0.0)


def _bn_call(feats_t, gamma3, beta3):
    return pl.pallas_call(
        _bn_body,
        out_shape=jax.ShapeDtypeStruct(feats_t.shape, jnp.float32),
    )(feats_t, gamma3, beta3)


def kernel(xyz, feature, weightbank, sW1, sb1, sW2, sb2, gamma, beta):
    B, N, _ = xyz.shape
    C = feature.shape[1]
    M = N // 4
    OUT = weightbank.shape[1] // _MM

    xyz_t = xyz.transpose(0, 2, 1).reshape(B, 3, 8, N // 8)
    featT_flat = feature.transpose(0, 2, 1).reshape(B * N, C)

    fidx, px, py, pz = _fps_call(xyz_t)
    p2 = jnp.stack([px, py, pz], axis=-1)

    featM_flat = _sc_gather1(featT_flat, fidx.reshape(B * M))
    featM = featM_flat.reshape(B, M, C)
    featM_T = featM.transpose(0, 2, 1)

    kg, km = _knn_call(featM, featM_T)
    gf_flat, fnb_flat = _sc_gather2(featT_flat, kg.reshape(-1),
                                    featM_flat, km.reshape(-1))
    gf = gf_flat.reshape(B, _K, M, C)
    fnb = fnb_flat.reshape(B, _K, M, C)

    feats_t = _dense_call(gf, fnb, sW1, sb1.reshape(1, -1), sW2,
                          sb2.reshape(1, -1), weightbank)
    out = _bn_call(feats_t, gamma.reshape(1, OUT, 1),
                   beta.reshape(1, OUT, 1))
    return (p2, out)
